# baseline (speedup 1.0000x reference)
import jax
import jax.numpy as jnp
from jax import lax
from jax.experimental import pallas as pl
from jax.experimental.pallas import tpu as pltpu, tpu_sc as plsc

EMB = 64
NC, NS = 2, 16
NW = NC * NS
BW = 128
NG = 8
NT = 4


def _build(n_batch, seq):
    n_bt = n_batch // BW
    bt_per_w = n_bt // NW
    n_blocks = seq * bt_per_w
    n_groups = n_blocks // NG
    mesh = plsc.VectorSubcoreMesh(core_axis_name="c", subcore_axis_name="s")

    def body(x_hbm, table_hbm, out_hbm, xv, *bufs):
        gbuf = bufs[:NG]
        tbuf = bufs[NG:NG + NT]
        sem_g = bufs[NG + NT:2 * NG + NT]
        sem_o = bufs[2 * NG + NT:]
        wid = lax.axis_index("s") * NC + lax.axis_index("c")
        bt0 = wid * bt_per_w

        pltpu.sync_copy(x_hbm.at[:, pl.ds(bt0, bt_per_w)], xv)

        def lj(g, q):
            return ((NG // bt_per_w) * g + q // bt_per_w, q % bt_per_w)

        def gather_copy(l, j, q):
            return pltpu.make_async_copy(
                table_hbm.at[xv.at[l, j]], gbuf[q], sem_g[q]
            )

        def store_copy(l, j, q):
            return pltpu.make_async_copy(
                tbuf[q % NT], out_hbm.at[l, :, bt0 + j], sem_o[q % NT]
            )

        rows = [lax.iota(jnp.int32, 16) + k * 16 for k in range(BW // 16)]

        def transpose(q):
            @plsc.parallel_loop(0, EMB, unroll=8)
            def _(e):
                cols = jnp.full((16,), e, jnp.int32)
                et = e // 8
                ei = e % 8
                for k in range(BW // 16):
                    vec = plsc.load_gather(gbuf[q], [rows[k], cols])
                    tbuf[q % NT][et, ei, pl.ds(k * 16, 16)] = vec

        for q in range(NG):
            l, j = lj(0, q)
            gather_copy(l, j, q).start()

        def group(g, carry):
            for q in range(NG):
                l, j = lj(g, q)
                gather_copy(l, j, q).wait()

                if q >= NT:
                    store_copy(*lj(g, q - NT), q - NT).wait()
                else:

                    @pl.when(g > 0)
                    def _():
                        store_copy(*lj(g - 1, q + NT), q + NT).wait()

                transpose(q)

                @pl.when(g < n_groups - 1)
                def _():
                    ln, jn = lj(g + 1, q)
                    gather_copy(ln, jn, q).start()

                store_copy(l, j, q).start()
            return carry

        lax.fori_loop(0, n_groups, group, 0)

        for q in range(NG - NT, NG):
            store_copy(*lj(n_groups - 1, q), q).wait()

    return pl.kernel(
        body,
        out_type=jax.ShapeDtypeStruct(
            (seq, EMB // 8, n_bt, 8, BW), jnp.float32
        ),
        mesh=mesh,
        scratch_types=[pltpu.VMEM((seq, bt_per_w, BW), jnp.int32)]
        + [pltpu.VMEM((BW, EMB), jnp.float32)] * NG
        + [pltpu.VMEM((EMB // 8, 8, BW), jnp.float32)] * NT
        + [pltpu.SemaphoreType.DMA] * NG
        + [pltpu.SemaphoreType.DMA] * NT,
        compiler_params=pltpu.CompilerParams(
            use_tc_tiling_on_sc=False, needs_layout_passes=False
        ),
    )


def kernel(x, table):
    B, L = x.shape
    x3 = jnp.transpose(x).reshape(L, B // BW, BW).astype(jnp.int32)
    out5 = _build(B, L)(x3, table)
    return out5.transpose(2, 4, 0, 1, 3).reshape(B, L, EMB)


# speedup vs baseline: 1.0251x; 1.0251x over previous
import jax
import jax.numpy as jnp
from jax import lax
from jax.experimental import pallas as pl
from jax.experimental.pallas import tpu as pltpu, tpu_sc as plsc

EMB = 64
NC, NS = 2, 16
NW = NC * NS
BW = 128
NG = 8
NT = 4


def _build(n_batch, seq):
    n_bt = n_batch // BW
    bt_per_w = n_bt // NW
    n_blocks = seq * bt_per_w
    n_groups = n_blocks // NG
    mesh = plsc.VectorSubcoreMesh(core_axis_name="c", subcore_axis_name="s")

    def body(x_hbm, table_hbm, out_hbm, xv, *bufs):
        gbuf = bufs[:NG]
        tbuf = bufs[NG:NG + NT]
        sem_g = bufs[NG + NT:2 * NG + NT]
        sem_o = bufs[2 * NG + NT:]
        wid = lax.axis_index("s") * NC + lax.axis_index("c")
        bt0 = wid * bt_per_w

        pltpu.sync_copy(x_hbm.at[:, pl.ds(bt0, bt_per_w)], xv)

        def lj(g, q):
            return ((NG // bt_per_w) * g + q // bt_per_w, q % bt_per_w)

        def gather_copy(l, j, q):
            return pltpu.make_async_copy(
                table_hbm.at[xv.at[l, j]], gbuf[q], sem_g[q]
            )

        def store_copy(l, j, q):
            return pltpu.make_async_copy(
                tbuf[q % NT], out_hbm.at[l, :, bt0 + j], sem_o[q % NT]
            )

        rows = [lax.iota(jnp.int32, 16) + k * 16 for k in range(BW // 16)]

        def transpose(q):
            @plsc.parallel_loop(0, EMB, unroll=4)
            def _(e):
                cols = jnp.full((16,), e, jnp.int32)
                et = e // 8
                ei = e % 8
                for k in range(BW // 16):
                    vec = plsc.load_gather(gbuf[q], [rows[k], cols])
                    tbuf[q % NT][et, ei, pl.ds(k * 16, 16)] = vec

        for q in range(NG):
            l, j = lj(0, q)
            gather_copy(l, j, q).start()

        def group(g, carry):
            for q in range(NG):
                l, j = lj(g, q)
                gather_copy(l, j, q).wait()

                if q >= NT:
                    store_copy(*lj(g, q - NT), q - NT).wait()
                else:

                    @pl.when(g > 0)
                    def _():
                        store_copy(*lj(g - 1, q + NT), q + NT).wait()

                transpose(q)

                @pl.when(g < n_groups - 1)
                def _():
                    ln, jn = lj(g + 1, q)
                    gather_copy(ln, jn, q).start()

                store_copy(l, j, q).start()
            return carry

        lax.fori_loop(0, n_groups, group, 0)

        for q in range(NG - NT, NG):
            store_copy(*lj(n_groups - 1, q), q).wait()

    return pl.kernel(
        body,
        out_type=jax.ShapeDtypeStruct(
            (seq, EMB // 8, n_bt, 8, BW), jnp.float32
        ),
        mesh=mesh,
        scratch_types=[pltpu.VMEM((seq, bt_per_w, BW), jnp.int32)]
        + [pltpu.VMEM((BW, EMB), jnp.float32)] * NG
        + [pltpu.VMEM((EMB // 8, 8, BW), jnp.float32)] * NT
        + [pltpu.SemaphoreType.DMA] * NG
        + [pltpu.SemaphoreType.DMA] * NT,
        compiler_params=pltpu.CompilerParams(
            use_tc_tiling_on_sc=False, needs_layout_passes=False
        ),
    )


def kernel(x, table):
    B, L = x.shape
    x3 = jnp.transpose(x).reshape(L, B // BW, BW).astype(jnp.int32)
    out5 = _build(B, L)(x3, table)
    return out5.transpose(2, 4, 0, 1, 3).reshape(B, L, EMB)
